# baseline (speedup 1.0000x reference)
import functools

import jax
import jax.numpy as jnp
from jax import lax
from jax.experimental import pallas as pl
from jax.experimental.pallas import tpu as pltpu
from jax.experimental.pallas import tpu_sc as plsc

N = 10000
E = 320000
D = 128

NC = 2
NS = 16
NW = NC * NS
C = 128
NBUF = 2
HC = 40
CPWA = 40
CPWB = 120
NCHUNK = NS * (CPWA + CPWB)
EPAD = NCHUNK * C
CD = 128
CPWD = 80
NCHUNKD = NW * CPWD
EPADD = NCHUNKD * CD
RPT = 632
ROWS = NS * RPT
RBLK = ROWS // 8


def _worker_ids():
    cid = lax.axis_index("c")
    sid = lax.axis_index("s")
    return cid, sid, sid * NC + cid


def _sc_agg_body(h, gch, sch, z128, accp, gidx, sidx, *rest):
    rows = rest[:NBUF]
    acc_sm = rest[NBUF]
    gsems = rest[NBUF + 1 : NBUF + 1 + NBUF]
    cid, sid, wid = _worker_ids()
    base = sid * RPT

    pltpu.sync_copy(z128, acc_sm.at[pl.ds(base, RPT)])
    plsc.subcore_barrier()

    mycpw = jnp.where(cid == 0, CPWA, CPWB)
    mybase = jnp.where(cid == 0, sid * CPWA, NS * CPWA + sid * CPWB)

    def slab_body(s, carry):
        s0 = mybase + s * HC
        pltpu.sync_copy(gch.at[pl.ds(s0, HC)], gidx)
        pltpu.sync_copy(sch.at[pl.ds(s0, HC)], sidx)
        for b in range(NBUF):
            pltpu.async_copy(h.at[gidx.at[b]], rows[b], gsems[b])

        def ring(i, c2):
            j = i * NBUF
            for b in range(NBUF):
                jb = j + b
                pltpu.make_async_copy(h.at[gidx.at[jb]], rows[b], gsems[b]).wait()
                pltpu.sync_copy(rows[b], acc_sm.at[sidx.at[jb]], add=True)

                @pl.when(jb + NBUF < HC)
                def _():
                    pltpu.async_copy(h.at[gidx.at[jb + NBUF]], rows[b], gsems[b])

            return c2

        lax.fori_loop(0, HC // NBUF, ring, 0)
        return carry

    lax.fori_loop(0, mycpw // HC, slab_body, 0)
    plsc.subcore_barrier()

    pltpu.sync_copy(acc_sm.at[pl.ds(base, RPT)], accp.at[cid, pl.ds(base, RPT)])


def _sc_deg_body(sch, z128, ones_w, degp, sidx, ones_v, deg_sm):
    cid, sid, wid = _worker_ids()
    base = sid * RPT

    pltpu.sync_copy(z128, deg_sm.at[pl.ds(base, RPT)])
    pltpu.sync_copy(ones_w, ones_v)
    pltpu.sync_copy(sch.at[pl.ds(wid * CPWD, CPWD)], sidx)
    plsc.subcore_barrier()

    def chunk(j, carry):
        pltpu.sync_copy(ones_v, deg_sm.at[sidx.at[j]], add=True)
        return carry

    lax.fori_loop(0, CPWD, chunk, 0)
    plsc.subcore_barrier()

    pltpu.sync_copy(deg_sm.at[pl.ds(base, RPT)], degp.at[cid, pl.ds(base, RPT)])


def _sc_mesh():
    return plsc.VectorSubcoreMesh(
        core_axis_name="c", subcore_axis_name="s", num_cores=NC, num_subcores=NS
    )


@functools.cache
def _make_sc_agg():
    return pl.kernel(
        _sc_agg_body,
        out_type=[jax.ShapeDtypeStruct((NC, ROWS, D), jnp.float32)],
        mesh=_sc_mesh(),
        scratch_types=[
            pltpu.VMEM((HC, C), jnp.int32),
            pltpu.VMEM((HC, C), jnp.int32),
            *[pltpu.VMEM((C, D), jnp.float32) for _ in range(NBUF)],
            pltpu.VMEM_SHARED((ROWS, D), jnp.float32),
            *[pltpu.SemaphoreType.DMA for _ in range(NBUF)],
        ],
        name="gcn_sc_agg",
    )


@functools.cache
def _make_sc_deg():
    return pl.kernel(
        _sc_deg_body,
        out_type=[jax.ShapeDtypeStruct((NC, ROWS, D), jnp.float32)],
        mesh=_sc_mesh(),
        scratch_types=[
            pltpu.VMEM((CPWD, CD), jnp.int32),
            pltpu.VMEM((CD, D), jnp.float32),
            pltpu.VMEM_SHARED((ROWS, D), jnp.float32),
        ],
        name="gcn_sc_deg",
    )


def _mm_body(x_ref, w_ref, b_ref, o_ref):
    o_ref[...] = (
        jnp.dot(x_ref[...], w_ref[...], preferred_element_type=jnp.float32)
        + b_ref[...][None, :]
    )


_tc_matmul = pl.pallas_call(
    _mm_body,
    grid=(8,),
    in_specs=[
        pl.BlockSpec((RBLK, D), lambda i: (i, 0)),
        pl.BlockSpec((D, D), lambda i: (0, 0)),
        pl.BlockSpec((D,), lambda i: (0,)),
    ],
    out_specs=pl.BlockSpec((RBLK, D), lambda i: (i, 0)),
    out_shape=jax.ShapeDtypeStruct((ROWS, D), jnp.float32),
)


def _mid_body(acc_ref, deg_ref, h_ref, w_ref, b_ref, o_ref):
    s = acc_ref[0] + acc_ref[1] + h_ref[...]
    d = (deg_ref[0] + deg_ref[1]) + 1.0
    g = jnp.maximum(s / d, 0.0)
    o_ref[...] = (
        jnp.dot(g, w_ref[...], preferred_element_type=jnp.float32)
        + b_ref[...][None, :]
    )


_tc_mid = pl.pallas_call(
    _mid_body,
    grid=(8,),
    in_specs=[
        pl.BlockSpec((NC, RBLK, D), lambda i: (0, i, 0)),
        pl.BlockSpec((NC, RBLK, D), lambda i: (0, i, 0)),
        pl.BlockSpec((RBLK, D), lambda i: (i, 0)),
        pl.BlockSpec((D, D), lambda i: (0, 0)),
        pl.BlockSpec((D,), lambda i: (0,)),
    ],
    out_specs=pl.BlockSpec((RBLK, D), lambda i: (i, 0)),
    out_shape=jax.ShapeDtypeStruct((ROWS, D), jnp.float32),
)


def _final_body(acc_ref, deg_ref, h_ref, o_ref):
    s = acc_ref[0] + acc_ref[1] + h_ref[...]
    d = (deg_ref[0] + deg_ref[1]) + 1.0
    o_ref[...] = s / d


_tc_final = pl.pallas_call(
    _final_body,
    grid=(8,),
    in_specs=[
        pl.BlockSpec((NC, RBLK, D), lambda i: (0, i, 0)),
        pl.BlockSpec((NC, RBLK, D), lambda i: (0, i, 0)),
        pl.BlockSpec((RBLK, D), lambda i: (i, 0)),
    ],
    out_specs=pl.BlockSpec((RBLK, D), lambda i: (i, 0)),
    out_shape=jax.ShapeDtypeStruct((ROWS, D), jnp.float32),
)


def kernel(x, edge_index, W1, b1, W2, b2):
    src = edge_index[0]
    dst = edge_index[1]
    pad = jnp.full((EPAD - E,), N, jnp.int32)
    srcf = jnp.concatenate([src, pad])
    dstf = jnp.concatenate([dst, pad])
    srcc = srcf.reshape(NCHUNK, C)
    dstc = dstf.reshape(NCHUNK, C)
    srcd = srcf.reshape(NCHUNKD, CD)
    dstd = dstf.reshape(NCHUNKD, CD)
    x_pad = jnp.pad(x, ((0, ROWS - N), (0, 0)))

    z128 = jnp.zeros((RPT, D), jnp.float32)
    ones_w = jnp.ones((CD, D), jnp.float32)

    (deg1,) = _make_sc_deg()(dstd, z128, ones_w)
    (deg2,) = _make_sc_deg()(srcd, z128, ones_w)
    h1 = _tc_matmul(x_pad, W1, b1)
    (acc1,) = _make_sc_agg()(h1, srcc, dstc, z128)
    h2 = _tc_mid(acc1, deg1, h1, W2, b2)
    (acc2,) = _make_sc_agg()(h2, dstc, srcc, z128)
    out = _tc_final(acc2, deg2, h2)
    return out[:N]


# speedup vs baseline: 1.1030x; 1.1030x over previous
import functools

import jax
import jax.numpy as jnp
from jax import lax
from jax.experimental import pallas as pl
from jax.experimental.pallas import tpu as pltpu
from jax.experimental.pallas import tpu_sc as plsc

N = 10000
E = 320000
D = 128

NC = 2
NS = 16
NW = NC * NS
C = 128
NBUF = 2
HC = 40
CPWA = 120
CPWB = 40
NCHUNK = NS * (CPWA + CPWB)
EPAD = NCHUNK * C
CD = 128
CPWD = 80
NCHUNKD = NW * CPWD
EPADD = NCHUNKD * CD
RPT = 632
ROWS = NS * RPT
RBLK = ROWS // 8


def _worker_ids():
    cid = lax.axis_index("c")
    sid = lax.axis_index("s")
    return cid, sid, sid * NC + cid


def _sc_agg_body(h, gch, sch, z128, accp, gidx, sidx, *rest):
    rows = rest[:NBUF]
    acc_sm = rest[NBUF]
    gsems = rest[NBUF + 1 : NBUF + 1 + NBUF]
    cid, sid, wid = _worker_ids()
    base = sid * RPT

    pltpu.sync_copy(z128, acc_sm.at[pl.ds(base, RPT)])
    plsc.subcore_barrier()

    mycpw = jnp.where(cid == 0, CPWA, CPWB)
    mybase = jnp.where(cid == 0, sid * CPWA, NS * CPWA + sid * CPWB)

    def slab_body(s, carry):
        s0 = mybase + s * HC
        pltpu.sync_copy(gch.at[pl.ds(s0, HC)], gidx)
        pltpu.sync_copy(sch.at[pl.ds(s0, HC)], sidx)
        for b in range(NBUF):
            pltpu.async_copy(h.at[gidx.at[b]], rows[b], gsems[b])

        def ring(i, c2):
            j = i * NBUF
            for b in range(NBUF):
                jb = j + b
                pltpu.make_async_copy(h.at[gidx.at[jb]], rows[b], gsems[b]).wait()
                pltpu.sync_copy(rows[b], acc_sm.at[sidx.at[jb]], add=True)

                @pl.when(jb + NBUF < HC)
                def _():
                    pltpu.async_copy(h.at[gidx.at[jb + NBUF]], rows[b], gsems[b])

            return c2

        lax.fori_loop(0, HC // NBUF, ring, 0)
        return carry

    lax.fori_loop(0, mycpw // HC, slab_body, 0)
    plsc.subcore_barrier()

    pltpu.sync_copy(acc_sm.at[pl.ds(base, RPT)], accp.at[cid, pl.ds(base, RPT)])


def _sc_deg_body(sch, z128, ones_w, degp, sidx, ones_v, deg_sm):
    cid, sid, wid = _worker_ids()
    base = sid * RPT

    pltpu.sync_copy(z128, deg_sm.at[pl.ds(base, RPT)])
    pltpu.sync_copy(ones_w, ones_v)
    pltpu.sync_copy(sch.at[pl.ds(wid * CPWD, CPWD)], sidx)
    plsc.subcore_barrier()

    def chunk(j, carry):
        pltpu.sync_copy(ones_v, deg_sm.at[sidx.at[j]], add=True)
        return carry

    lax.fori_loop(0, CPWD, chunk, 0)
    plsc.subcore_barrier()

    pltpu.sync_copy(deg_sm.at[pl.ds(base, RPT)], degp.at[cid, pl.ds(base, RPT)])


def _sc_mesh():
    return plsc.VectorSubcoreMesh(
        core_axis_name="c", subcore_axis_name="s", num_cores=NC, num_subcores=NS
    )


@functools.cache
def _make_sc_agg():
    return pl.kernel(
        _sc_agg_body,
        out_type=[jax.ShapeDtypeStruct((NC, ROWS, D), jnp.float32)],
        mesh=_sc_mesh(),
        scratch_types=[
            pltpu.VMEM((HC, C), jnp.int32),
            pltpu.VMEM((HC, C), jnp.int32),
            *[pltpu.VMEM((C, D), jnp.float32) for _ in range(NBUF)],
            pltpu.VMEM_SHARED((ROWS, D), jnp.float32),
            *[pltpu.SemaphoreType.DMA for _ in range(NBUF)],
        ],
        name="gcn_sc_agg",
    )


@functools.cache
def _make_sc_deg():
    return pl.kernel(
        _sc_deg_body,
        out_type=[jax.ShapeDtypeStruct((NC, ROWS, D), jnp.float32)],
        mesh=_sc_mesh(),
        scratch_types=[
            pltpu.VMEM((CPWD, CD), jnp.int32),
            pltpu.VMEM((CD, D), jnp.float32),
            pltpu.VMEM_SHARED((ROWS, D), jnp.float32),
        ],
        name="gcn_sc_deg",
    )


def _mm_body(x_ref, w_ref, b_ref, o_ref):
    o_ref[...] = (
        jnp.dot(x_ref[...], w_ref[...], preferred_element_type=jnp.float32)
        + b_ref[...][None, :]
    )


_tc_matmul = pl.pallas_call(
    _mm_body,
    grid=(8,),
    in_specs=[
        pl.BlockSpec((RBLK, D), lambda i: (i, 0)),
        pl.BlockSpec((D, D), lambda i: (0, 0)),
        pl.BlockSpec((D,), lambda i: (0,)),
    ],
    out_specs=pl.BlockSpec((RBLK, D), lambda i: (i, 0)),
    out_shape=jax.ShapeDtypeStruct((ROWS, D), jnp.float32),
)


def _mid_body(acc_ref, deg_ref, h_ref, w_ref, b_ref, o_ref):
    s = acc_ref[0] + acc_ref[1] + h_ref[...]
    d = (deg_ref[0] + deg_ref[1]) + 1.0
    g = jnp.maximum(s / d, 0.0)
    o_ref[...] = (
        jnp.dot(g, w_ref[...], preferred_element_type=jnp.float32)
        + b_ref[...][None, :]
    )


_tc_mid = pl.pallas_call(
    _mid_body,
    grid=(8,),
    in_specs=[
        pl.BlockSpec((NC, RBLK, D), lambda i: (0, i, 0)),
        pl.BlockSpec((NC, RBLK, D), lambda i: (0, i, 0)),
        pl.BlockSpec((RBLK, D), lambda i: (i, 0)),
        pl.BlockSpec((D, D), lambda i: (0, 0)),
        pl.BlockSpec((D,), lambda i: (0,)),
    ],
    out_specs=pl.BlockSpec((RBLK, D), lambda i: (i, 0)),
    out_shape=jax.ShapeDtypeStruct((ROWS, D), jnp.float32),
)


def _final_body(acc_ref, deg_ref, h_ref, o_ref):
    s = acc_ref[0] + acc_ref[1] + h_ref[...]
    d = (deg_ref[0] + deg_ref[1]) + 1.0
    o_ref[...] = s / d


_tc_final = pl.pallas_call(
    _final_body,
    grid=(8,),
    in_specs=[
        pl.BlockSpec((NC, RBLK, D), lambda i: (0, i, 0)),
        pl.BlockSpec((NC, RBLK, D), lambda i: (0, i, 0)),
        pl.BlockSpec((RBLK, D), lambda i: (i, 0)),
    ],
    out_specs=pl.BlockSpec((RBLK, D), lambda i: (i, 0)),
    out_shape=jax.ShapeDtypeStruct((ROWS, D), jnp.float32),
)


def kernel(x, edge_index, W1, b1, W2, b2):
    src = edge_index[0]
    dst = edge_index[1]
    pad = jnp.full((EPAD - E,), N, jnp.int32)
    srcf = jnp.concatenate([src, pad])
    dstf = jnp.concatenate([dst, pad])
    srcc = srcf.reshape(NCHUNK, C)
    dstc = dstf.reshape(NCHUNK, C)
    srcd = srcf.reshape(NCHUNKD, CD)
    dstd = dstf.reshape(NCHUNKD, CD)
    x_pad = jnp.pad(x, ((0, ROWS - N), (0, 0)))

    z128 = jnp.zeros((RPT, D), jnp.float32)
    ones_w = jnp.ones((CD, D), jnp.float32)

    (deg1,) = _make_sc_deg()(dstd, z128, ones_w)
    (deg2,) = _make_sc_deg()(srcd, z128, ones_w)
    h1 = _tc_matmul(x_pad, W1, b1)
    (acc1,) = _make_sc_agg()(h1, srcc, dstc, z128)
    h2 = _tc_mid(acc1, deg1, h1, W2, b2)
    (acc2,) = _make_sc_agg()(h2, dstc, srcc, z128)
    out = _tc_final(acc2, deg2, h2)
    return out[:N]
